# baseline (speedup 1.0000x reference)
import jax
import jax.numpy as jnp
import numpy as np
from jax import lax
from jax.experimental import pallas as pl
from jax.experimental.pallas import tpu as pltpu
from jax.experimental.pallas import tpu_sc as plsc

N = 10000
E = 320000
D_IN = 128
D_EDGE = 16
D_OUT = 128
HEADS = 4
DH = D_OUT // HEADS

NC = 2
NS = 16
NW = NC * NS
EPW = E // NW
C = 80
NCH = EPW // C
NP = 10240
RPT = NP // NS

_f32 = jnp.float32
_bf16 = jnp.bfloat16


def _sc_mesh():
    return plsc.VectorSubcoreMesh(
        core_axis_name="c", subcore_axis_name="s", num_cores=NC, num_subcores=NS
    )


def _gather_body(x_hbm, src_hbm, dst_hbm, xs_hbm, xd_hbm,
                 idxs_v, idxd_v, xsb, xdb, sem_s, sem_d):
    c = lax.axis_index("c")
    s = lax.axis_index("s")
    wid = s * NC + c

    def chunk(j, carry):
        base = wid * EPW + j * C
        pltpu.sync_copy(src_hbm.at[pl.ds(base, C)], idxs_v)
        pltpu.sync_copy(dst_hbm.at[pl.ds(base, C)], idxd_v)
        cp1 = pltpu.async_copy(x_hbm.at[idxs_v], xsb, sem_s)
        cp2 = pltpu.async_copy(x_hbm.at[idxd_v], xdb, sem_d)
        cp1.wait()
        cp2.wait()
        pltpu.sync_copy(xsb, xs_hbm.at[pl.ds(base, C)])
        pltpu.sync_copy(xdb, xd_hbm.at[pl.ds(base, C)])
        return carry

    lax.fori_loop(0, NCH, chunk, 0)


def _gather_call(x, src, dst):
    return pl.kernel(
        _gather_body,
        out_type=(
            jax.ShapeDtypeStruct((E, D_IN), _f32),
            jax.ShapeDtypeStruct((E, D_IN), _f32),
        ),
        mesh=_sc_mesh(),
        scratch_types=[
            pltpu.VMEM((C,), jnp.int32),
            pltpu.VMEM((C,), jnp.int32),
            pltpu.VMEM((C, D_IN), _f32),
            pltpu.VMEM((C, D_IN), _f32),
            pltpu.SemaphoreType.DMA,
            pltpu.SemaphoreType.DMA,
        ],
    )(x, src, dst)


def _msg_body(dst_hbm, v_hbm, z_hbm, agg_hbm, idx_v, vb, agg_sh):
    c = lax.axis_index("c")
    s = lax.axis_index("s")
    wid = s * NC + c
    pltpu.sync_copy(z_hbm.at[pl.ds(s * RPT, RPT)], agg_sh.at[pl.ds(s * RPT, RPT)])
    plsc.subcore_barrier()

    def chunk(j, carry):
        base = wid * EPW + j * C
        pltpu.sync_copy(dst_hbm.at[pl.ds(base, C)], idx_v)
        pltpu.sync_copy(v_hbm.at[pl.ds(base, C)], vb)
        pltpu.sync_copy(vb, agg_sh.at[idx_v], add=True)
        return carry

    lax.fori_loop(0, NCH, chunk, 0)
    plsc.subcore_barrier()
    pltpu.sync_copy(agg_sh.at[pl.ds(s * RPT, RPT)],
                    agg_hbm.at[c, pl.ds(s * RPT, RPT)])


def _msg_call(dst, v, zn128):
    return pl.kernel(
        _msg_body,
        out_type=jax.ShapeDtypeStruct((NC, NP, D_OUT), _f32),
        mesh=_sc_mesh(),
        scratch_types=[
            pltpu.VMEM((C,), jnp.int32),
            pltpu.VMEM((C, D_OUT), _f32),
            pltpu.VMEM_SHARED((NP, D_OUT), _f32),
        ],
    )(dst, v, zn128)


def _denom_body(dst_hbm, ex_hbm, z_hbm, den_hbm, idx_v, exb, exb128, den_sh):
    c = lax.axis_index("c")
    s = lax.axis_index("s")
    wid = s * NC + c
    pltpu.sync_copy(z_hbm.at[pl.ds(s * RPT, RPT)], den_sh.at[pl.ds(s * RPT, RPT)])

    def zrow(e, carry):
        for r in range(1, D_OUT // 16):
            exb128[e, pl.ds(r * 16, 16)] = jnp.zeros((16,), _f32)
        return carry

    lax.fori_loop(0, C, zrow, 0)
    plsc.subcore_barrier()

    def chunk(j, carry):
        base = wid * EPW + j * C
        pltpu.sync_copy(dst_hbm.at[pl.ds(base, C)], idx_v)
        pltpu.sync_copy(ex_hbm.at[pl.ds(base, C)], exb)

        def edge(e, carry2):
            exb128[e, pl.ds(0, 16)] = exb[e, :]
            return carry2

        lax.fori_loop(0, C, edge, 0)
        pltpu.sync_copy(exb128, den_sh.at[idx_v], add=True)
        return carry

    lax.fori_loop(0, NCH, chunk, 0)
    plsc.subcore_barrier()
    pltpu.sync_copy(den_sh.at[pl.ds(s * RPT, RPT)],
                    den_hbm.at[c, pl.ds(s * RPT, RPT)])


def _denom_call(dst, ex16, zn128):
    return pl.kernel(
        _denom_body,
        out_type=jax.ShapeDtypeStruct((NC, NP, D_OUT), _f32),
        mesh=_sc_mesh(),
        scratch_types=[
            pltpu.VMEM((C,), jnp.int32),
            pltpu.VMEM((C, 16), _f32),
            pltpu.VMEM((C, D_OUT), _f32),
            pltpu.VMEM_SHARED((NP, D_OUT), _f32),
        ],
    )(dst, ex16, zn128)


_EB = 1000


def _dense_body(xs_ref, xd_ref, ea_ref, wka, wkb, wkc, kb0r, pp, cv,
                wva, wvb, wvc, vb0r, mv, vb1r, ex_ref, v_ref):
    xs = xs_ref[...].astype(_bf16)
    xd = xd_ref[...].astype(_bf16)
    ea = ea_ref[...].astype(_bf16)
    h0k = jnp.maximum(
        jnp.dot(xs, wka[...], preferred_element_type=_f32)
        + jnp.dot(xd, wkb[...], preferred_element_type=_f32)
        + jnp.dot(ea, wkc[...], preferred_element_type=_f32)
        + kb0r[...], 0.0)
    ex_ref[...] = jnp.exp(
        jnp.dot(h0k, pp[...], preferred_element_type=_f32) + cv[...])
    h0v = jnp.maximum(
        jnp.dot(xs, wva[...], preferred_element_type=_f32)
        + jnp.dot(xd, wvb[...], preferred_element_type=_f32)
        + jnp.dot(ea, wvc[...], preferred_element_type=_f32)
        + vb0r[...], 0.0)
    ve = jnp.dot(h0v.astype(_bf16), mv[...], preferred_element_type=_f32) + vb1r[...]
    exf = ex_ref[...]
    v_ref[...] = ve * jnp.repeat(exf[:, :HEADS], DH, axis=1)


def _dense_call(xs, xd, ea, wka, wkb, wkc, kb0r, pp, cv, wva, wvb, wvc,
                vb0r, mv, vb1r):
    full = lambda r, k: pl.BlockSpec((r, k), lambda i: (0, 0))
    return pl.pallas_call(
        _dense_body,
        grid=(E // _EB,),
        in_specs=[
            pl.BlockSpec((_EB, D_IN), lambda i: (i, 0)),
            pl.BlockSpec((_EB, D_IN), lambda i: (i, 0)),
            pl.BlockSpec((_EB, D_EDGE), lambda i: (i, 0)),
            full(D_IN, D_OUT), full(D_IN, D_OUT), full(D_EDGE, D_OUT),
            full(1, D_OUT), full(D_OUT, 16), full(1, 16),
            full(D_IN, D_OUT), full(D_IN, D_OUT), full(D_EDGE, D_OUT),
            full(1, D_OUT), full(D_OUT, D_OUT), full(1, D_OUT),
        ],
        out_specs=[
            pl.BlockSpec((_EB, 16), lambda i: (i, 0)),
            pl.BlockSpec((_EB, D_OUT), lambda i: (i, 0)),
        ],
        out_shape=[
            jax.ShapeDtypeStruct((E, 16), _f32),
            jax.ShapeDtypeStruct((E, D_OUT), _f32),
        ],
    )(xs, xd, ea, wka, wkb, wkc, kb0r, pp, cv, wva, wvb, wvc, vb0r, mv, vb1r)


_NB = 1000


def _final_body(agg_ref, den_ref, x_ref, ow0, ob0r, ow1, ob1r, out_ref):
    u = agg_ref[0] + agg_ref[1]
    den = den_ref[0] + den_ref[1]
    inv = 1.0 / (den[:, :HEADS] + 1e-16)
    agg = u * jnp.repeat(inv, DH, axis=1)
    h = jnp.maximum(agg, 0.0)
    h = jnp.maximum(
        jnp.dot(h, ow0[...], preferred_element_type=_f32) + ob0r[...], 0.0)
    h = h + jnp.dot(h, ow1[...], preferred_element_type=_f32) + ob1r[...]
    out_ref[...] = jnp.maximum(x_ref[...] + h, 0.0)


def _final_call(agg, den, x, ow0, ob0r, ow1, ob1r):
    full = lambda r, k: pl.BlockSpec((r, k), lambda i: (0, 0))
    return pl.pallas_call(
        _final_body,
        grid=(N // _NB,),
        in_specs=[
            pl.BlockSpec((NC, _NB, D_OUT), lambda i: (0, i, 0)),
            pl.BlockSpec((NC, _NB, D_OUT), lambda i: (0, i, 0)),
            pl.BlockSpec((_NB, D_IN), lambda i: (i, 0)),
            full(D_OUT, D_OUT), full(1, D_OUT),
            full(D_OUT, D_OUT), full(1, D_OUT),
        ],
        out_specs=pl.BlockSpec((_NB, D_OUT), lambda i: (i, 0)),
        out_shape=jax.ShapeDtypeStruct((N, D_OUT), _f32),
    )(agg, den, x, ow0, ob0r, ow1, ob1r)


def kernel(x, edge_attr, edge_index, q, k_W0, k_b0, k_W1, k_b1,
           v_W0, v_b0, v_W1, v_b1, o_W0, o_b0, o_W1, o_b1):
    src = edge_index[0]
    dst = edge_index[1]

    wka, wkb, wkc = k_W0[:D_IN], k_W0[D_IN:2 * D_IN], k_W0[2 * D_IN:]
    wva, wvb, wvc = v_W0[:D_IN], v_W0[D_IN:2 * D_IN], v_W0[2 * D_IN:]
    eye = jnp.eye(D_OUT, dtype=_f32)
    qv = q[0]
    qm = jnp.zeros((D_OUT, HEADS), _f32).at[
        jnp.arange(D_OUT), jnp.arange(D_OUT) // DH].set(qv)
    sc = np.float32(1.0 / np.sqrt(DH))
    pp = (eye + k_W1) @ qm * sc
    cv = (k_b1 @ qm) * sc
    pp16 = jnp.concatenate([pp, jnp.zeros((D_OUT, 12), _f32)], axis=1)
    cv16 = jnp.concatenate([cv, jnp.zeros((12,), _f32)]).reshape(1, 16)
    mv = eye + v_W1

    zn128 = jnp.zeros((NP, D_OUT), _f32)

    xs, xd = _gather_call(x, src, dst)
    ex16, v = _dense_call(
        xs, xd, edge_attr,
        wka.astype(_bf16), wkb.astype(_bf16), wkc.astype(_bf16),
        k_b0.reshape(1, -1), pp16, cv16,
        wva.astype(_bf16), wvb.astype(_bf16), wvc.astype(_bf16),
        v_b0.reshape(1, -1), mv.astype(_bf16), v_b1.reshape(1, -1))
    agg = _msg_call(dst, v, zn128)
    den = _denom_call(dst, ex16, zn128)
    return _final_call(agg, den, x, o_W0, o_b0.reshape(1, -1), o_W1,
                       o_b1.reshape(1, -1))


# speedup vs baseline: 1.4349x; 1.4349x over previous
import jax
import jax.numpy as jnp
import numpy as np
from jax import lax
from jax.experimental import pallas as pl
from jax.experimental.pallas import tpu as pltpu
from jax.experimental.pallas import tpu_sc as plsc

N = 10000
E = 320000
D_IN = 128
D_EDGE = 16
D_OUT = 128
HEADS = 4
DH = D_OUT // HEADS

NC = 2
NS = 16
NW = NC * NS
C = 80
NP = 10240
RPT = NP // NS

RANGES = ((0, 128000), (128000, 128000), (256000, 64000))

_f32 = jnp.float32


def _sc_mesh():
    return plsc.VectorSubcoreMesh(
        core_axis_name="c", subcore_axis_name="s", num_cores=NC, num_subcores=NS
    )


def _make_gather(base0, er):
    epw = er // NW
    nch = epw // C

    def body(x_hbm, src_hbm, dst_hbm, xs_hbm, xd_hbm,
             idxs_v, idxd_v, xsb, xdb, sem_s, sem_d):
        c = lax.axis_index("c")
        s = lax.axis_index("s")
        wid = s * NC + c

        def chunk(j, carry):
            lbase = wid * epw + j * C
            gbase = base0 + lbase
            pltpu.sync_copy(src_hbm.at[pl.ds(gbase, C)], idxs_v)
            pltpu.sync_copy(dst_hbm.at[pl.ds(gbase, C)], idxd_v)
            cp1 = pltpu.async_copy(x_hbm.at[idxs_v], xsb, sem_s)
            cp2 = pltpu.async_copy(x_hbm.at[idxd_v], xdb, sem_d)
            cp1.wait()
            cp2.wait()
            pltpu.sync_copy(xsb, xs_hbm.at[pl.ds(lbase, C)])
            pltpu.sync_copy(xdb, xd_hbm.at[pl.ds(lbase, C)])
            return carry

        lax.fori_loop(0, nch, chunk, 0)

    return pl.kernel(
        body,
        out_type=(
            jax.ShapeDtypeStruct((er, D_IN), _f32),
            jax.ShapeDtypeStruct((er, D_IN), _f32),
        ),
        mesh=_sc_mesh(),
        scratch_types=[
            pltpu.VMEM((C,), jnp.int32),
            pltpu.VMEM((C,), jnp.int32),
            pltpu.VMEM((C, D_IN), _f32),
            pltpu.VMEM((C, D_IN), _f32),
            pltpu.SemaphoreType.DMA,
            pltpu.SemaphoreType.DMA,
        ],
    )


def _make_msg(base0, er):
    epw = er // NW
    nch = epw // C

    def body(dst_hbm, v_hbm, z_hbm, agg_hbm, idx_v, vb, agg_sh):
        c = lax.axis_index("c")
        s = lax.axis_index("s")
        wid = s * NC + c
        pltpu.sync_copy(z_hbm.at[pl.ds(s * RPT, RPT)],
                        agg_sh.at[pl.ds(s * RPT, RPT)])
        plsc.subcore_barrier()

        def chunk(j, carry):
            lbase = wid * epw + j * C
            pltpu.sync_copy(dst_hbm.at[pl.ds(base0 + lbase, C)], idx_v)
            pltpu.sync_copy(v_hbm.at[pl.ds(lbase, C)], vb)
            pltpu.sync_copy(vb, agg_sh.at[idx_v], add=True)
            return carry

        lax.fori_loop(0, nch, chunk, 0)
        plsc.subcore_barrier()
        pltpu.sync_copy(agg_sh.at[pl.ds(s * RPT, RPT)],
                        agg_hbm.at[c, pl.ds(s * RPT, RPT)])

    return pl.kernel(
        body,
        out_type=jax.ShapeDtypeStruct((NC, NP, D_OUT), _f32),
        mesh=_sc_mesh(),
        scratch_types=[
            pltpu.VMEM((C,), jnp.int32),
            pltpu.VMEM((C, D_OUT), _f32),
            pltpu.VMEM_SHARED((NP, D_OUT), _f32),
        ],
    )


def _make_denom(base0, er):
    epw = er // NW
    nch = epw // C

    def body(dst_hbm, ex_hbm, z_hbm, den_hbm, idx_v, exb, exb128, den_sh):
        c = lax.axis_index("c")
        s = lax.axis_index("s")
        wid = s * NC + c
        pltpu.sync_copy(z_hbm.at[pl.ds(s * RPT, RPT)],
                        den_sh.at[pl.ds(s * RPT, RPT)])

        def zrow(e, carry):
            for r in range(1, D_OUT // 16):
                exb128[e, pl.ds(r * 16, 16)] = jnp.zeros((16,), _f32)
            return carry

        lax.fori_loop(0, C, zrow, 0)
        plsc.subcore_barrier()

        def chunk(j, carry):
            lbase = wid * epw + j * C
            pltpu.sync_copy(dst_hbm.at[pl.ds(base0 + lbase, C)], idx_v)
            pltpu.sync_copy(ex_hbm.at[pl.ds(lbase, C)], exb)

            def edge(e, carry2):
                exb128[e, pl.ds(0, 16)] = exb[e, :]
                return carry2

            lax.fori_loop(0, C, edge, 0)
            pltpu.sync_copy(exb128, den_sh.at[idx_v], add=True)
            return carry

        lax.fori_loop(0, nch, chunk, 0)
        plsc.subcore_barrier()
        pltpu.sync_copy(den_sh.at[pl.ds(s * RPT, RPT)],
                        den_hbm.at[c, pl.ds(s * RPT, RPT)])

    return pl.kernel(
        body,
        out_type=jax.ShapeDtypeStruct((NC, NP, D_OUT), _f32),
        mesh=_sc_mesh(),
        scratch_types=[
            pltpu.VMEM((C,), jnp.int32),
            pltpu.VMEM((C, 16), _f32),
            pltpu.VMEM((C, D_OUT), _f32),
            pltpu.VMEM_SHARED((NP, D_OUT), _f32),
        ],
    )


_EB = 1000


def _dense_body(xs_ref, xd_ref, ea_ref, wka, wkb, wkc, kb0r, pp, cv,
                wva, wvb, wvc, vb0r, mv, vb1r, ex_ref, v_ref):
    xs = xs_ref[...]
    xd = xd_ref[...]
    ea = ea_ref[...]
    h0k = jnp.maximum(
        jnp.dot(xs, wka[...], preferred_element_type=_f32)
        + jnp.dot(xd, wkb[...], preferred_element_type=_f32)
        + jnp.dot(ea, wkc[...], preferred_element_type=_f32)
        + kb0r[...], 0.0)
    ex_ref[...] = jnp.exp(
        jnp.dot(h0k, pp[...], preferred_element_type=_f32) + cv[...])
    h0v = jnp.maximum(
        jnp.dot(xs, wva[...], preferred_element_type=_f32)
        + jnp.dot(xd, wvb[...], preferred_element_type=_f32)
        + jnp.dot(ea, wvc[...], preferred_element_type=_f32)
        + vb0r[...], 0.0)
    ve = jnp.dot(h0v, mv[...], preferred_element_type=_f32) + vb1r[...]
    exf = ex_ref[...]
    v_ref[...] = ve * jnp.repeat(exf[:, :HEADS], DH, axis=1)


def _dense_call(base0, er, xs, xd, ea, wka, wkb, wkc, kb0r, pp, cv,
                wva, wvb, wvc, vb0r, mv, vb1r):
    offb = base0 // _EB
    full = lambda r, k: pl.BlockSpec((r, k), lambda i: (0, 0))
    return pl.pallas_call(
        _dense_body,
        grid=(er // _EB,),
        in_specs=[
            pl.BlockSpec((_EB, D_IN), lambda i: (i, 0)),
            pl.BlockSpec((_EB, D_IN), lambda i: (i, 0)),
            pl.BlockSpec((_EB, D_EDGE), lambda i, o=offb: (i + o, 0)),
            full(D_IN, D_OUT), full(D_IN, D_OUT), full(D_EDGE, D_OUT),
            full(1, D_OUT), full(D_OUT, 16), full(1, 16),
            full(D_IN, D_OUT), full(D_IN, D_OUT), full(D_EDGE, D_OUT),
            full(1, D_OUT), full(D_OUT, D_OUT), full(1, D_OUT),
        ],
        out_specs=[
            pl.BlockSpec((_EB, 16), lambda i: (i, 0)),
            pl.BlockSpec((_EB, D_OUT), lambda i: (i, 0)),
        ],
        out_shape=[
            jax.ShapeDtypeStruct((er, 16), _f32),
            jax.ShapeDtypeStruct((er, D_OUT), _f32),
        ],
    )(xs, xd, ea, wka, wkb, wkc, kb0r, pp, cv, wva, wvb, wvc, vb0r, mv, vb1r)


_NB = 1000


def _final_body(a0_ref, a1_ref, a2_ref, d0_ref, d1_ref, d2_ref,
                x_ref, ow0, ob0r, ow1, ob1r, out_ref):
    u = (a0_ref[0] + a0_ref[1] + a1_ref[0] + a1_ref[1]
         + a2_ref[0] + a2_ref[1])
    den = (d0_ref[0] + d0_ref[1] + d1_ref[0] + d1_ref[1]
           + d2_ref[0] + d2_ref[1])
    inv = 1.0 / (den[:, :HEADS] + 1e-16)
    agg = u * jnp.repeat(inv, DH, axis=1)
    h = jnp.maximum(agg, 0.0)
    h = jnp.maximum(
        jnp.dot(h, ow0[...], preferred_element_type=_f32) + ob0r[...], 0.0)
    h = h + jnp.dot(h, ow1[...], preferred_element_type=_f32) + ob1r[...]
    out_ref[...] = jnp.maximum(x_ref[...] + h, 0.0)


def _final_call(aggs, dens, x, ow0, ob0r, ow1, ob1r):
    full = lambda r, k: pl.BlockSpec((r, k), lambda i: (0, 0))
    part = pl.BlockSpec((NC, _NB, D_OUT), lambda i: (0, i, 0))
    return pl.pallas_call(
        _final_body,
        grid=(N // _NB,),
        in_specs=[
            part, part, part, part, part, part,
            pl.BlockSpec((_NB, D_IN), lambda i: (i, 0)),
            full(D_OUT, D_OUT), full(1, D_OUT),
            full(D_OUT, D_OUT), full(1, D_OUT),
        ],
        out_specs=pl.BlockSpec((_NB, D_OUT), lambda i: (i, 0)),
        out_shape=jax.ShapeDtypeStruct((N, D_OUT), _f32),
    )(*aggs, *dens, x, ow0, ob0r, ow1, ob1r)


def kernel(x, edge_attr, edge_index, q, k_W0, k_b0, k_W1, k_b1,
           v_W0, v_b0, v_W1, v_b1, o_W0, o_b0, o_W1, o_b1):
    src = edge_index[0]
    dst = edge_index[1]

    wka, wkb, wkc = k_W0[:D_IN], k_W0[D_IN:2 * D_IN], k_W0[2 * D_IN:]
    wva, wvb, wvc = v_W0[:D_IN], v_W0[D_IN:2 * D_IN], v_W0[2 * D_IN:]
    eye = jnp.eye(D_OUT, dtype=_f32)
    qv = q[0]
    qm = jnp.zeros((D_OUT, HEADS), _f32).at[
        jnp.arange(D_OUT), jnp.arange(D_OUT) // DH].set(qv)
    sc = np.float32(1.0 / np.sqrt(DH))
    pp = (eye + k_W1) @ qm * sc
    cv = (k_b1 @ qm) * sc
    pp16 = jnp.concatenate([pp, jnp.zeros((D_OUT, 12), _f32)], axis=1)
    cv16 = jnp.concatenate([cv, jnp.zeros((12,), _f32)]).reshape(1, 16)
    mv = eye + v_W1

    zn128 = jnp.zeros((NP, D_OUT), _f32)

    aggs, dens = [], []
    for base0, er in RANGES:
        xs, xd = _make_gather(base0, er)(x, src, dst)
        ex16, v = _dense_call(
            base0, er, xs, xd, edge_attr, wka, wkb, wkc,
            k_b0.reshape(1, -1), pp16, cv16,
            wva, wvb, wvc, v_b0.reshape(1, -1), mv, v_b1.reshape(1, -1))
        aggs.append(_make_msg(base0, er)(dst, v, zn128))
        dens.append(_make_denom(base0, er)(dst, ex16, zn128))
    return _final_call(aggs, dens, x, o_W0, o_b0.reshape(1, -1), o_W1,
                       o_b1.reshape(1, -1))
